# baseline (speedup 1.0000x reference)
import jax
import jax.numpy as jnp
from jax import lax
from jax.experimental import pallas as pl
from jax.experimental.pallas import tpu as pltpu
from jax.experimental.pallas import tpu_sc as plsc

DIM = 128
Q = 32
NQV = Q // 16
KDIM = 3
E = 160000
N_NODES = 10000
N_SUB = 16
CHUNK = 80
EDGES_PER_TILE = E // N_SUB
NCHUNK = EDGES_PER_TILE // CHUNK
ROWS_PER_TILE = N_NODES // N_SUB
EBLK = 2000
NBLK = EDGES_PER_TILE // EBLK
GPB = EBLK // 16


def _sc_body(vec_h, p1_h, p2_h, ev_h, ei_h, out_h,
             vec0, vec1, msg0, msg1, p10, p11, p20, p21, ev0, ev1,
             eib, dsbuf,
             sin0, sin1, ssc0, ssc1, sg0, sg1, so0, so1, agg):
    c = lax.axis_index("c")
    s = lax.axis_index("s")
    ebase = s * EDGES_PER_TILE
    vecb = (vec0, vec1)
    msgb = (msg0, msg1)
    p1b = (p10, p11)
    p2b = (p20, p21)
    evb = (ev0, ev1)
    sin = (sin0, sin1)
    ssc = (ssc0, ssc1)
    sg = (sg0, sg1)
    so = (so0, so1)
    iota16 = lax.iota(jnp.int32, 16)

    def load_indices(col):
        colv = jnp.full((16,), col, jnp.int32)
        for h in range(NBLK):
            pltpu.sync_copy(ei_h.at[pl.ds(ebase + h * EBLK, EBLK), :], eib)

            def dg(g, cc):
                vals = plsc.load_gather(eib, [iota16 + 16 * g, colv])
                f = h * EBLK + 16 * g
                dsbuf[f // CHUNK, pl.ds(f % CHUNK, 16)] = vals
                return cc
            lax.fori_loop(0, GPB, dg, 0)

    def pass_body(p, carry):
        qoff = p * (2 * Q) + c * Q

        load_indices(1)

        def zrow(buf):
            def zr(r, cc):
                for k in range(KDIM):
                    for i in range(NQV):
                        buf[r, k, pl.ds(16 * i, 16)] = jnp.zeros((16,),
                                                                 jnp.float32)
                return cc
            lax.fori_loop(0, CHUNK, zr, 0)
        zrow(msg0)
        zrow(msg1)
        rbase = s * ROWS_PER_TILE
        nfull = ROWS_PER_TILE // CHUNK
        for j in range(nfull):
            pltpu.sync_copy(msg0, agg.at[pl.ds(rbase + j * CHUNK, CHUNK)])
        rem = ROWS_PER_TILE % CHUNK
        if rem:
            pltpu.sync_copy(msg0.at[pl.ds(0, rem)],
                            agg.at[pl.ds(rbase + nfull * CHUNK, rem)])
        plsc.subcore_barrier()

        def fetch_start(b, j):
            eb = ebase + j * CHUNK
            pltpu.async_copy(vec_h.at[pl.ds(eb, CHUNK), :, pl.ds(qoff, Q)],
                             vecb[b], sin[b])
            pltpu.async_copy(p1_h.at[pl.ds(eb, CHUNK), pl.ds(qoff, Q)],
                             p1b[b], sin[b])
            pltpu.async_copy(p2_h.at[pl.ds(eb, CHUNK), pl.ds(qoff, Q)],
                             p2b[b], sin[b])
            pltpu.async_copy(ev_h.at[pl.ds(eb, CHUNK), :],
                             evb[b].at[:, pl.ds(0, KDIM)], sin[b])

        def fetch_wait(b):
            pltpu.make_async_copy(vec_h.at[pl.ds(0, CHUNK), :, pl.ds(0, Q)],
                                  vecb[b], sin[b]).wait()
            pltpu.make_async_copy(p1_h.at[pl.ds(0, CHUNK), pl.ds(0, Q)],
                                  p1b[b], sin[b]).wait()
            pltpu.make_async_copy(p2_h.at[pl.ds(0, CHUNK), pl.ds(0, Q)],
                                  p2b[b], sin[b]).wait()
            pltpu.make_async_copy(ev_h.at[pl.ds(0, CHUNK), :],
                                  evb[b].at[:, pl.ds(0, KDIM)], sin[b]).wait()

        def scat_start(b, j):
            pltpu.async_copy(msgb[b], agg.at[dsbuf.at[j]], ssc[b], add=True)

        def scat_wait(b):
            pltpu.make_async_copy(msgb[b], agg.at[dsbuf.at[0]], ssc[b]).wait()

        def compute(b):
            def edge_body(e, ec):
                p1v = [p1b[b][e, pl.ds(16 * i, 16)] for i in range(NQV)]
                p2v = [p2b[b][e, pl.ds(16 * i, 16)] for i in range(NQV)]
                evv = evb[b][e, pl.ds(0, 16)]
                for k in range(KDIM):
                    evk = evv[k]
                    for i in range(NQV):
                        msgb[b][e, k, pl.ds(16 * i, 16)] = (
                            vecb[b][e, k, pl.ds(16 * i, 16)] * p1v[i]
                            + p2v[i] * evk)
                return ec
            lax.fori_loop(0, CHUNK, edge_body, 0, unroll=4)

        def sc_iter(b, j):
            fetch_wait(b)
            scat_wait(b)
            compute(b)
            @pl.when(j + 2 < NCHUNK)
            def _():
                fetch_start(b, j + 2)
            scat_start(b, j)

        fetch_start(0, 0)
        fetch_start(1, 1)
        pltpu.async_copy(msg0, agg.at[dsbuf.at[0]], ssc[0], add=True)
        pltpu.async_copy(msg1, agg.at[dsbuf.at[0]], ssc[1], add=True)

        def loop_body(jj, cc):
            sc_iter(0, 2 * jj)
            sc_iter(1, 2 * jj + 1)
            return cc
        lax.fori_loop(0, NCHUNK // 2, loop_body, 0)
        sc_iter(0, NCHUNK - 1)
        scat_wait(0)
        scat_wait(1)
        plsc.subcore_barrier()

        load_indices(0)
        gB = (msg0, msg1)

        def g_start(b, j):
            pltpu.async_copy(agg.at[dsbuf.at[j]], gB[b], sg[b])

        def g_wait(b):
            pltpu.make_async_copy(agg.at[dsbuf.at[0]], gB[b], sg[b]).wait()

        def w_start(b, j):
            eb = ebase + j * CHUNK
            pltpu.async_copy(gB[b],
                             out_h.at[0, pl.ds(eb, CHUNK), :, pl.ds(qoff, Q)],
                             so[b])

        def w_wait(b):
            pltpu.make_async_copy(gB[b],
                                  out_h.at[0, pl.ds(0, CHUNK), :, pl.ds(0, Q)],
                                  so[b]).wait()

        g_start(0, 0)
        g_wait(0)
        g_start(1, 1)
        w_start(0, 0)

        def g_iter(b, j, guard):
            g_wait(b)
            w_wait(1 - b)
            if guard:
                @pl.when(j + 1 < NCHUNK)
                def _():
                    g_start(1 - b, j + 1)
            else:
                g_start(1 - b, j + 1)
            w_start(b, j)

        def g_body(jj, cc):
            g_iter(1, 2 * jj + 1, False)
            g_iter(0, 2 * jj + 2, True)
            return cc
        lax.fori_loop(0, (NCHUNK - 1) // 2, g_body, 0)
        w_wait(0)
        plsc.subcore_barrier()
        return carry

    lax.fori_loop(0, 2, pass_body, 0)


@jax.jit
def _mpnn_sc(vec, p1, p2, ev, ei):
    mesh = plsc.VectorSubcoreMesh(core_axis_name="c", subcore_axis_name="s")
    f32 = jnp.float32
    run = pl.kernel(
        _sc_body,
        mesh=mesh,
        out_type=jax.ShapeDtypeStruct((1, E, KDIM, DIM), f32),
        scratch_types=[
            pltpu.VMEM((CHUNK, KDIM, Q), f32),
            pltpu.VMEM((CHUNK, KDIM, Q), f32),
            pltpu.VMEM((CHUNK, KDIM, Q), f32),
            pltpu.VMEM((CHUNK, KDIM, Q), f32),
            pltpu.VMEM((CHUNK, Q), f32),
            pltpu.VMEM((CHUNK, Q), f32),
            pltpu.VMEM((CHUNK, Q), f32),
            pltpu.VMEM((CHUNK, Q), f32),
            pltpu.VMEM((CHUNK, 16), f32),
            pltpu.VMEM((CHUNK, 16), f32),
            pltpu.VMEM((EBLK, 2), jnp.int32),
            pltpu.VMEM((NCHUNK, CHUNK), jnp.int32),
            pltpu.SemaphoreType.DMA,
            pltpu.SemaphoreType.DMA,
            pltpu.SemaphoreType.DMA,
            pltpu.SemaphoreType.DMA,
            pltpu.SemaphoreType.DMA,
            pltpu.SemaphoreType.DMA,
            pltpu.SemaphoreType.DMA,
            pltpu.SemaphoreType.DMA,
            pltpu.VMEM_SHARED((N_NODES, KDIM, Q), f32),
        ],
        compiler_params=pltpu.CompilerParams(use_tc_tiling_on_sc=False,
                                             needs_layout_passes=False),
    )
    return run(vec, p1, p2, ev, ei)


def kernel(edge_index, vec, pos_vec1_list, pos_vec2_list, edge_vec):
    p1 = pos_vec1_list.reshape(E, DIM)
    p2 = pos_vec2_list.reshape(E, DIM)
    ev = edge_vec.reshape(E, KDIM)
    return _mpnn_sc(vec, p1, p2, ev, edge_index)


# speedup vs baseline: 1.0781x; 1.0781x over previous
import jax
import jax.numpy as jnp
from jax import lax
from jax.experimental import pallas as pl
from jax.experimental.pallas import tpu as pltpu
from jax.experimental.pallas import tpu_sc as plsc

DIM = 128
Q = 32
NQV = Q // 16
KDIM = 3
E = 160000
N_NODES = 10000
N_SUB = 16
CHUNK = 80
EDGES_PER_TILE = E // N_SUB
NCHUNK = EDGES_PER_TILE // CHUNK
ROWS_PER_TILE = N_NODES // N_SUB


def _sc_body(vec_h, p1_h, p2_h, ev_h, src_h, dst_h, out_h,
             vec0, vec1, msg0, msg1, p10, p11, p20, p21, ev0, ev1, dsbuf,
             sin0, sin1, ssc0, ssc1, sg0, sg1, so0, so1, agg):
    c = lax.axis_index("c")
    s = lax.axis_index("s")
    ebase = s * EDGES_PER_TILE
    vecb = (vec0, vec1)
    msgb = (msg0, msg1)
    p1b = (p10, p11)
    p2b = (p20, p21)
    evb = (ev0, ev1)
    sin = (sin0, sin1)
    ssc = (ssc0, ssc1)
    sg = (sg0, sg1)
    so = (so0, so1)

    def pass_body(p, carry):
        qoff = p * (2 * Q) + c * Q

        pltpu.sync_copy(dst_h.at[s], dsbuf)

        def zrow(buf):
            def zr(r, cc):
                for k in range(KDIM):
                    for i in range(NQV):
                        buf[r, k, pl.ds(16 * i, 16)] = jnp.zeros((16,),
                                                                 jnp.float32)
                return cc
            lax.fori_loop(0, CHUNK, zr, 0)
        zrow(msg0)
        zrow(msg1)
        rbase = s * ROWS_PER_TILE
        nfull = ROWS_PER_TILE // CHUNK
        for j in range(nfull):
            pltpu.sync_copy(msg0, agg.at[pl.ds(rbase + j * CHUNK, CHUNK)])
        rem = ROWS_PER_TILE % CHUNK
        if rem:
            pltpu.sync_copy(msg0.at[pl.ds(0, rem)],
                            agg.at[pl.ds(rbase + nfull * CHUNK, rem)])
        plsc.subcore_barrier()

        def fetch_start(b, j):
            eb = ebase + j * CHUNK
            pltpu.async_copy(vec_h.at[pl.ds(eb, CHUNK), :, pl.ds(qoff, Q)],
                             vecb[b], sin[b])
            pltpu.async_copy(p1_h.at[pl.ds(eb, CHUNK), pl.ds(qoff, Q)],
                             p1b[b], sin[b])
            pltpu.async_copy(p2_h.at[pl.ds(eb, CHUNK), pl.ds(qoff, Q)],
                             p2b[b], sin[b])
            pltpu.async_copy(ev_h.at[pl.ds(eb, CHUNK), :],
                             evb[b].at[:, pl.ds(0, KDIM)], sin[b])

        def fetch_wait(b):
            pltpu.make_async_copy(vec_h.at[pl.ds(0, CHUNK), :, pl.ds(0, Q)],
                                  vecb[b], sin[b]).wait()
            pltpu.make_async_copy(p1_h.at[pl.ds(0, CHUNK), pl.ds(0, Q)],
                                  p1b[b], sin[b]).wait()
            pltpu.make_async_copy(p2_h.at[pl.ds(0, CHUNK), pl.ds(0, Q)],
                                  p2b[b], sin[b]).wait()
            pltpu.make_async_copy(ev_h.at[pl.ds(0, CHUNK), :],
                                  evb[b].at[:, pl.ds(0, KDIM)], sin[b]).wait()

        def scat_start(b, j):
            pltpu.async_copy(msgb[b], agg.at[dsbuf.at[j]], ssc[b], add=True)

        def scat_wait(b):
            pltpu.make_async_copy(msgb[b], agg.at[dsbuf.at[0]], ssc[b]).wait()

        def compute(b):
            def edge_body(e, ec):
                p1v = [p1b[b][e, pl.ds(16 * i, 16)] for i in range(NQV)]
                p2v = [p2b[b][e, pl.ds(16 * i, 16)] for i in range(NQV)]
                evv = evb[b][e, pl.ds(0, 16)]
                for k in range(KDIM):
                    evk = evv[k]
                    for i in range(NQV):
                        msgb[b][e, k, pl.ds(16 * i, 16)] = (
                            vecb[b][e, k, pl.ds(16 * i, 16)] * p1v[i]
                            + p2v[i] * evk)
                return ec
            lax.fori_loop(0, CHUNK, edge_body, 0, unroll=4)

        def sc_iter(b, j):
            fetch_wait(b)
            scat_wait(b)
            compute(b)
            @pl.when(j + 2 < NCHUNK)
            def _():
                fetch_start(b, j + 2)
            scat_start(b, j)

        fetch_start(0, 0)
        fetch_start(1, 1)
        pltpu.async_copy(msg0, agg.at[dsbuf.at[0]], ssc[0], add=True)
        pltpu.async_copy(msg1, agg.at[dsbuf.at[0]], ssc[1], add=True)

        def loop_body(jj, cc):
            sc_iter(0, 2 * jj)
            sc_iter(1, 2 * jj + 1)
            return cc
        lax.fori_loop(0, NCHUNK // 2, loop_body, 0)
        sc_iter(0, NCHUNK - 1)
        scat_wait(0)
        scat_wait(1)
        plsc.subcore_barrier()

        pltpu.sync_copy(src_h.at[s], dsbuf)
        gB = (msg0, msg1)

        def g_start(b, j):
            pltpu.async_copy(agg.at[dsbuf.at[j]], gB[b], sg[b])

        def g_wait(b):
            pltpu.make_async_copy(agg.at[dsbuf.at[0]], gB[b], sg[b]).wait()

        def w_start(b, j):
            eb = ebase + j * CHUNK
            pltpu.async_copy(gB[b],
                             out_h.at[0, pl.ds(eb, CHUNK), :, pl.ds(qoff, Q)],
                             so[b])

        def w_wait(b):
            pltpu.make_async_copy(gB[b],
                                  out_h.at[0, pl.ds(0, CHUNK), :, pl.ds(0, Q)],
                                  so[b]).wait()

        g_start(0, 0)
        g_wait(0)
        g_start(1, 1)
        w_start(0, 0)

        def g_iter(b, j, guard):
            g_wait(b)
            w_wait(1 - b)
            if guard:
                @pl.when(j + 1 < NCHUNK)
                def _():
                    g_start(1 - b, j + 1)
            else:
                g_start(1 - b, j + 1)
            w_start(b, j)

        def g_body(jj, cc):
            g_iter(1, 2 * jj + 1, False)
            g_iter(0, 2 * jj + 2, True)
            return cc
        lax.fori_loop(0, (NCHUNK - 1) // 2, g_body, 0)
        w_wait(0)
        plsc.subcore_barrier()
        return carry

    lax.fori_loop(0, 2, pass_body, 0)


@jax.jit
def _mpnn_sc(vec, p1, p2, ev, src2, dst2):
    mesh = plsc.VectorSubcoreMesh(core_axis_name="c", subcore_axis_name="s")
    f32 = jnp.float32
    run = pl.kernel(
        _sc_body,
        mesh=mesh,
        out_type=jax.ShapeDtypeStruct((1, E, KDIM, DIM), f32),
        scratch_types=[
            pltpu.VMEM((CHUNK, KDIM, Q), f32),
            pltpu.VMEM((CHUNK, KDIM, Q), f32),
            pltpu.VMEM((CHUNK, KDIM, Q), f32),
            pltpu.VMEM((CHUNK, KDIM, Q), f32),
            pltpu.VMEM((CHUNK, Q), f32),
            pltpu.VMEM((CHUNK, Q), f32),
            pltpu.VMEM((CHUNK, Q), f32),
            pltpu.VMEM((CHUNK, Q), f32),
            pltpu.VMEM((CHUNK, 16), f32),
            pltpu.VMEM((CHUNK, 16), f32),
            pltpu.VMEM((NCHUNK, CHUNK), jnp.int32),
            pltpu.SemaphoreType.DMA,
            pltpu.SemaphoreType.DMA,
            pltpu.SemaphoreType.DMA,
            pltpu.SemaphoreType.DMA,
            pltpu.SemaphoreType.DMA,
            pltpu.SemaphoreType.DMA,
            pltpu.SemaphoreType.DMA,
            pltpu.SemaphoreType.DMA,
            pltpu.VMEM_SHARED((N_NODES, KDIM, Q), f32),
        ],
        compiler_params=pltpu.CompilerParams(use_tc_tiling_on_sc=False),
    )
    return run(vec, p1, p2, ev, src2, dst2)


def kernel(edge_index, vec, pos_vec1_list, pos_vec2_list, edge_vec):
    src2 = edge_index[:, 0].reshape(N_SUB, NCHUNK, CHUNK)
    dst2 = edge_index[:, 1].reshape(N_SUB, NCHUNK, CHUNK)
    p1 = pos_vec1_list.reshape(E, DIM)
    p2 = pos_vec2_list.reshape(E, DIM)
    ev = edge_vec.reshape(E, KDIM)
    return _mpnn_sc(vec, p1, p2, ev, src2, dst2)
